# baseline (speedup 1.0000x reference)
import functools

import jax
import jax.numpy as jnp
from jax import lax
from jax.experimental import pallas as pl
from jax.experimental.pallas import tpu as pltpu
from jax.experimental.pallas import tpu_sc as plsc

ENT_TOT = 1000000
REL_TOT = 1000
B = 16384
DIM = 64
NC = 2
NS = 16
NW = NC * NS
ROWS_PER_W = B // NW
CHUNK = 128
NCHUNK = ROWS_PER_W // CHUNK
GROUPS = CHUNK // 16
JBLOCK = 8

_F32 = jnp.float32
_MAGIC = 0x5F3759DF


def _rsqrt(x):
    i = plsc.bitcast(x, jnp.int32)
    y = plsc.bitcast(jnp.int32(_MAGIC) - (i >> 1), _F32)
    for _ in range(3):
        y = y * (_F32(1.5) - _F32(0.5) * x * y * y)
    return y


def _sqrt(x):
    return x * _rsqrt(jnp.maximum(x, _F32(1e-30)))


def _sc_body(head_hbm, rel_hbm, tail_hbm, negv_hbm, ent_t, rel_t,
             pos_out, neg_out, dist_out,
             idx_h, idx_r, idx_t, idx_n,
             h_buf, r_buf, t_buf, n_buf,
             pos_b, neg1_b, neg2_b, dist_b, sem):
    cid = lax.axis_index("c")
    sid = lax.axis_index("s")
    wid = sid * NC + cid
    base = wid * ROWS_PER_W
    zero = jnp.zeros((16,), _F32)

    def chunk_body(c, dist_acc):
        cbase = base + c * CHUNK
        pltpu.sync_copy(head_hbm.at[pl.ds(cbase, CHUNK)], idx_h.at[c])
        pltpu.sync_copy(rel_hbm.at[pl.ds(cbase, CHUNK)], idx_r.at[c])
        pltpu.sync_copy(tail_hbm.at[pl.ds(cbase, CHUNK)], idx_t.at[c])
        pltpu.sync_copy(negv_hbm.at[pl.ds(cbase, CHUNK)], idx_n.at[c])

        def gather_block(jb, carry):
            cps = []
            for u in range(JBLOCK):
                j = jb * JBLOCK + u
                cps.append(pltpu.async_copy(
                    ent_t.at[j].at[idx_h.at[c]], h_buf.at[j], sem))
                cps.append(pltpu.async_copy(
                    rel_t.at[j].at[idx_r.at[c]], r_buf.at[j], sem))
                cps.append(pltpu.async_copy(
                    ent_t.at[j].at[idx_t.at[c]], t_buf.at[j], sem))
                cps.append(pltpu.async_copy(
                    ent_t.at[j].at[idx_n.at[c]], n_buf.at[j], sem))
            for cp in cps:
                cp.wait()
            return carry

        lax.fori_loop(0, DIM // JBLOCK, gather_block, 0)

        def group_body(g, d_acc):
            gs = pl.ds(g * 16, 16)

            def norms(j, accs):
                sh_v, st_v, sn_v = accs
                hv = h_buf[j, gs]
                tv = t_buf[j, gs]
                nv = n_buf[j, gs]
                return sh_v + hv * hv, st_v + tv * tv, sn_v + nv * nv

            sh_v, st_v, sn_v = lax.fori_loop(0, DIM, norms,
                                             (zero, zero, zero))
            ihv = _rsqrt(jnp.maximum(sh_v, _F32(1e-24)))
            itv = _rsqrt(jnp.maximum(st_v, _F32(1e-24)))
            iqv = _rsqrt(jnp.maximum(sn_v, _F32(1e-24)))

            def scores(j, accs):
                sp_v, s1_v, s2_v, sd_v = accs
                hk = h_buf[j, gs]
                rk = r_buf[j, gs]
                tk = t_buf[j, gs]
                nk = n_buf[j, gs]
                hn = hk * ihv
                tn = tk * itv
                nn = nk * iqv
                cc = hn + rk
                bb = rk - tn
                pv = cc - tn
                n1 = bb + nn
                n2 = cc - nn
                dv = hk - tk
                return (sp_v + pv * pv, s1_v + n1 * n1,
                        s2_v + n2 * n2, sd_v + dv * dv)

            sp_v, s1_v, s2_v, sd_v = lax.fori_loop(0, DIM, scores,
                                                   (zero, zero, zero, zero))
            pos_b[gs] = -_sqrt(sp_v)
            neg1_b[gs] = -_sqrt(s1_v)
            neg2_b[gs] = -_sqrt(s2_v)
            return d_acc + _sqrt(sd_v)

        dist_acc = lax.fori_loop(0, GROUPS, group_body, dist_acc)

        pltpu.sync_copy(pos_b, pos_out.at[pl.ds(cbase, CHUNK)])
        pltpu.sync_copy(pos_b, pos_out.at[pl.ds(B + cbase, CHUNK)])
        pltpu.sync_copy(neg1_b, neg_out.at[pl.ds(cbase, CHUNK)])
        pltpu.sync_copy(neg2_b, neg_out.at[pl.ds(B + cbase, CHUNK)])
        return dist_acc

    dist_acc = lax.fori_loop(0, NCHUNK, chunk_body, zero)
    dist_b[...] = dist_acc
    pltpu.sync_copy(dist_b, dist_out.at[wid])


@functools.partial(jax.jit, static_argnames=())
def _sc_call(batch_head, batch_rel, batch_tail, batch_negative, ent_t, rel_t):
    mesh = plsc.VectorSubcoreMesh(core_axis_name="c", subcore_axis_name="s",
                                  num_cores=NC, num_subcores=NS)
    f = pl.kernel(
        _sc_body,
        out_type=(
            jax.ShapeDtypeStruct((2 * B,), _F32),
            jax.ShapeDtypeStruct((2 * B,), _F32),
            jax.ShapeDtypeStruct((NW, 16), _F32),
        ),
        mesh=mesh,
        compiler_params=pltpu.CompilerParams(needs_layout_passes=False,
                                             use_tc_tiling_on_sc=False),
        scratch_types=[
            pltpu.VMEM((NCHUNK, CHUNK), jnp.int32),
            pltpu.VMEM((NCHUNK, CHUNK), jnp.int32),
            pltpu.VMEM((NCHUNK, CHUNK), jnp.int32),
            pltpu.VMEM((NCHUNK, CHUNK), jnp.int32),
            pltpu.VMEM((DIM, CHUNK), _F32),
            pltpu.VMEM((DIM, CHUNK), _F32),
            pltpu.VMEM((DIM, CHUNK), _F32),
            pltpu.VMEM((DIM, CHUNK), _F32),
            pltpu.VMEM((CHUNK,), _F32),
            pltpu.VMEM((CHUNK,), _F32),
            pltpu.VMEM((CHUNK,), _F32),
            pltpu.VMEM((16,), _F32),
            pltpu.SemaphoreType.DMA,
        ],
    )
    return f(batch_head, batch_rel, batch_tail, batch_negative, ent_t, rel_t)


def kernel(batch_head, batch_rel, batch_tail, batch_negative, ent_emb, rel_emb):
    pos, neg, dist_parts = _sc_call(batch_head, batch_rel, batch_tail,
                                    batch_negative, ent_emb.T, rel_emb.T)
    return pos, neg, jnp.sum(dist_parts)
